# Initial kernel scaffold; baseline (speedup 1.0000x reference)
import jax
import jax.numpy as jnp
from jax.experimental import pallas as pl


def kernel(x, edge_index, edge_attr, W_ef, b_ef, W_eb, b_eb, W1, b1, W2, b2, W3, b3):
    raise NotImplementedError("write your pallas kernel here")


# speedup vs baseline: 2.5460x; 2.5460x over previous
import functools

import jax
import jax.numpy as jnp
from jax import lax
from jax.experimental import pallas as pl
from jax.experimental.pallas import tpu as pltpu
from jax.experimental.pallas import tpu_sc as plsc

H = 128
CHUNK = 128
NSUB = 16
LANES = 16


def _edge_linear_body(ea_ref, w_ref, b_ref, out_ref):
    a = ea_ref[...]
    out_ref[0] = (
        jnp.dot(a, w_ref[0], preferred_element_type=jnp.float32) + b_ref[0]
    )
    out_ref[1] = (
        jnp.dot(a, w_ref[1], preferred_element_type=jnp.float32) + b_ref[1]
    )


def _edge_linear(edge_attr, w_stack, b_stack, block_e=640):
    E = edge_attr.shape[0]
    return pl.pallas_call(
        _edge_linear_body,
        grid=(E // block_e,),
        in_specs=[
            pl.BlockSpec((block_e, H), lambda i: (i, 0)),
            pl.BlockSpec((2, H, H), lambda i: (0, 0, 0)),
            pl.BlockSpec((2, 1, H), lambda i: (0, 0, 0)),
        ],
        out_specs=pl.BlockSpec((2, block_e, H), lambda i: (0, i, 0)),
        out_shape=jax.ShapeDtypeStruct((2, E, H), jnp.float32),
    )(edge_attr, w_stack, b_stack)


def _sc_aggregate(edge_index, x, e_stack):
    N = x.shape[0]
    E = edge_index.shape[1]
    n_chunks = E // CHUNK
    chunks_per_sub = (n_chunks + NSUB - 1) // NSUB
    rows_per_sub = 624
    extra_rows = N - NSUB * rows_per_sub
    full_zero = rows_per_sub // CHUNK
    rem_zero = rows_per_sub % CHUNK

    mesh = plsc.VectorSubcoreMesh(core_axis_name="c", subcore_axis_name="s")

    @functools.partial(
        pl.kernel,
        out_type=jax.ShapeDtypeStruct((2, N, H), jnp.float32),
        mesh=mesh,
        scratch_types=[
            pltpu.VMEM((CHUNK,), jnp.int32),
            pltpu.VMEM((CHUNK,), jnp.int32),
            pltpu.VMEM((CHUNK, H), jnp.float32),
            pltpu.VMEM((CHUNK, H), jnp.float32),
            pltpu.VMEM_SHARED((N, H), jnp.float32),
            pltpu.SemaphoreType.DMA,
            pltpu.SemaphoreType.DMA,
        ],
    )
    def k(ei_hbm, x_hbm, e_hbm, out_hbm, gidx_v, sidx_v, xg_v, e_v, acc_sh,
          sem_e, sem_x):
        c = lax.axis_index("c")
        s = lax.axis_index("s")

        @pl.loop(0, CHUNK)
        def _(i):
            for j in range(H // LANES):
                xg_v[i, pl.ds(j * LANES, LANES)] = jnp.zeros(
                    (LANES,), jnp.float32
                )

        base_rows = s * rows_per_sub

        @pl.loop(0, full_zero)
        def _(t):
            pltpu.sync_copy(
                xg_v, acc_sh.at[pl.ds(base_rows + t * CHUNK, CHUNK)]
            )

        pltpu.sync_copy(
            xg_v.at[pl.ds(0, rem_zero)],
            acc_sh.at[pl.ds(base_rows + full_zero * CHUNK, rem_zero)],
        )

        @pl.when(s == NSUB - 1)
        def _():
            pltpu.sync_copy(
                xg_v.at[pl.ds(0, extra_rows)],
                acc_sh.at[pl.ds(NSUB * rows_per_sub, extra_rows)],
            )

        plsc.subcore_barrier()

        @pl.loop(0, chunks_per_sub)
        def _(t):
            ci = t * NSUB + s

            @pl.when(ci < n_chunks)
            def _():
                base = ci * CHUNK
                pltpu.sync_copy(ei_hbm.at[c, pl.ds(base, CHUNK)], gidx_v)
                pltpu.sync_copy(ei_hbm.at[1 - c, pl.ds(base, CHUNK)], sidx_v)
                cp_e = pltpu.async_copy(
                    e_hbm.at[c, pl.ds(base, CHUNK)], e_v, sem_e
                )
                cp_x = pltpu.async_copy(x_hbm.at[gidx_v], xg_v, sem_x)
                cp_e.wait()
                cp_x.wait()

                @pl.loop(0, CHUNK)
                def _(i):
                    for j in range(H // LANES):
                        sl = pl.ds(j * LANES, LANES)
                        xg_v[i, sl] = jnp.maximum(
                            xg_v[i, sl] + e_v[i, sl], 0.0
                        )

                pltpu.sync_copy(xg_v, acc_sh.at[sidx_v], add=True)

        plsc.subcore_barrier()
        pltpu.sync_copy(
            acc_sh.at[pl.ds(base_rows, rows_per_sub)],
            out_hbm.at[c, pl.ds(base_rows, rows_per_sub)],
        )

        @pl.when(s == NSUB - 1)
        def _():
            pltpu.sync_copy(
                acc_sh.at[pl.ds(NSUB * rows_per_sub, extra_rows)],
                out_hbm.at[c, pl.ds(NSUB * rows_per_sub, extra_rows)],
            )

    return k(edge_index, x, e_stack)


def _node_mlp_body(x_ref, agg_ref, w1_ref, b1_ref, w2_ref, b2_ref, w3_ref,
                   b3_ref, out_ref):
    w1 = w1_ref[...]
    b1 = b1_ref[...]
    w2 = w2_ref[...]
    b2 = b2_ref[...]

    def head(a):
        h = jnp.maximum(
            jnp.dot(a, w1, preferred_element_type=jnp.float32) + b1, 0.0
        )
        return jnp.dot(h, w2, preferred_element_type=jnp.float32) + b2

    yf = head(agg_ref[0])
    yb = head(agg_ref[1])
    xb = x_ref[...]
    out = (
        jnp.dot(xb, w3_ref[0:H], preferred_element_type=jnp.float32)
        + jnp.dot(yf, w3_ref[H:2 * H], preferred_element_type=jnp.float32)
        + jnp.dot(yb, w3_ref[2 * H:3 * H], preferred_element_type=jnp.float32)
        + b3_ref[...]
    )
    out_ref[...] = out


def _node_mlp(x, aggr, W1, b1, W2, b2, W3, b3, block_n=1000):
    N = x.shape[0]
    return pl.pallas_call(
        _node_mlp_body,
        grid=(N // block_n,),
        in_specs=[
            pl.BlockSpec((block_n, H), lambda i: (i, 0)),
            pl.BlockSpec((2, block_n, H), lambda i: (0, i, 0)),
            pl.BlockSpec((H, 2 * H), lambda i: (0, 0)),
            pl.BlockSpec((1, 2 * H), lambda i: (0, 0)),
            pl.BlockSpec((2 * H, H), lambda i: (0, 0)),
            pl.BlockSpec((1, H), lambda i: (0, 0)),
            pl.BlockSpec((3 * H, H), lambda i: (0, 0)),
            pl.BlockSpec((1, H), lambda i: (0, 0)),
        ],
        out_specs=pl.BlockSpec((block_n, H), lambda i: (i, 0)),
        out_shape=jax.ShapeDtypeStruct((N, H), jnp.float32),
    )(x, aggr, W1, b1.reshape(1, -1), W2, b2.reshape(1, -1), W3,
      b3.reshape(1, -1))


def kernel(x, edge_index, edge_attr, W_ef, b_ef, W_eb, b_eb, W1, b1, W2, b2,
           W3, b3):
    w_stack = jnp.stack([W_ef, W_eb])
    b_stack = jnp.stack([b_ef, b_eb]).reshape(2, 1, H)
    e_stack = _edge_linear(edge_attr, w_stack, b_stack)
    aggr = _sc_aggregate(edge_index, x, e_stack)
    return _node_mlp(x, aggr, W1, b1, W2, b2, W3, b3)
